# baseline (speedup 1.0000x reference)
import functools

import jax
import jax.numpy as jnp
from jax import lax
from jax.experimental import pallas as pl
from jax.experimental.pallas import tpu as pltpu
from jax.experimental.pallas import tpu_sc as plsc

N_FACTORS = 32
BATCH = 16384
PACK = 4
ROWS = 1000000 // PACK
NC = 2
NS = 16
NW = NC * NS
BPW = BATCH // NW
CHUNK = 128
NCHUNKS = BPW // CHUNK

_mesh = plsc.VectorSubcoreMesh(core_axis_name="c", subcore_axis_name="s")


@functools.partial(
    pl.kernel,
    out_type=jax.ShapeDtypeStruct((BATCH,), jnp.float32),
    mesh=_mesh,
    compiler_params=pltpu.CompilerParams(needs_layout_passes=False),
    scratch_types=[
        pltpu.VMEM((NCHUNKS, CHUNK), jnp.int32),
        pltpu.VMEM((NCHUNKS, CHUNK), jnp.int32),
        pltpu.VMEM((BPW,), jnp.int32),
        pltpu.VMEM((BPW,), jnp.int32),
        pltpu.VMEM((CHUNK, 4 * N_FACTORS), jnp.float32),
        pltpu.VMEM((CHUNK, 4 * N_FACTORS), jnp.float32),
        pltpu.VMEM((BPW,), jnp.float32),
        pltpu.SemaphoreType.DMA,
        pltpu.SemaphoreType.DMA,
    ],
)
def _sc_dot_kernel(urow_hbm, irow_hbm, uoff_hbm, ioff_hbm, u2_hbm, v2_hbm,
                   out_hbm, row_u, row_i, off_u, off_i, ubuf, vbuf, outv,
                   sem_u, sem_v):
    wid = lax.axis_index("s") * NC + lax.axis_index("c")
    base = wid * BPW
    pltpu.sync_copy(urow_hbm.at[pl.ds(wid * NCHUNKS, NCHUNKS)], row_u)
    pltpu.sync_copy(irow_hbm.at[pl.ds(wid * NCHUNKS, NCHUNKS)], row_i)
    pltpu.sync_copy(uoff_hbm.at[pl.ds(base, BPW)], off_u)
    pltpu.sync_copy(ioff_hbm.at[pl.ds(base, BPW)], off_i)
    lane = lax.iota(jnp.int32, 16)
    for c in range(NCHUNKS):
        cu = pltpu.async_copy(u2_hbm.at[row_u.at[c]], ubuf, sem_u)
        cv = pltpu.async_copy(v2_hbm.at[row_i.at[c]], vbuf, sem_v)
        cu.wait()
        cv.wait()

        def g_body(g, _, _c=c):
            gbase = g * 16
            lu = off_u[pl.ds(_c * CHUNK + gbase, 16)]
            li = off_i[pl.ds(_c * CHUNK + gbase, 16)]
            erow = gbase + lane
            acc = jnp.zeros((16,), jnp.float32)
            for d in range(N_FACTORS):
                uu = plsc.load_gather(ubuf, [erow, lu + d])
                vv = plsc.load_gather(vbuf, [erow, li + d])
                acc = acc + uu * vv
            outv[pl.ds(_c * CHUNK + gbase, 16)] = acc
            return 0

        lax.fori_loop(0, CHUNK // 16, g_body, 0)
    pltpu.sync_copy(outv, out_hbm.at[pl.ds(base, BPW)])


def kernel(data, user_factors, item_factors):
    users = data[:, 0].astype(jnp.int32)
    items = data[:, 1].astype(jnp.int32)
    urow = (users // PACK).reshape(NW * NCHUNKS, CHUNK)
    irow = (items // PACK).reshape(NW * NCHUNKS, CHUNK)
    uoff = (users % PACK) * N_FACTORS
    ioff = (items % PACK) * N_FACTORS
    u2 = user_factors.reshape(ROWS, PACK * N_FACTORS)
    v2 = item_factors.reshape(ROWS, PACK * N_FACTORS)
    return _sc_dot_kernel(urow, irow, uoff, ioff, u2, v2)


# speedup vs baseline: 3.3990x; 3.3990x over previous
import functools

import jax
import jax.numpy as jnp
from jax import lax
from jax.experimental import pallas as pl
from jax.experimental.pallas import tpu as pltpu
from jax.experimental.pallas import tpu_sc as plsc

N_FACTORS = 32
BATCH = 16384
NC = 2
NS = 16
NW = NC * NS
BPW = BATCH // NW
KBLK = 8
NBLK = BPW // KBLK

_mesh = plsc.VectorSubcoreMesh(core_axis_name="c", subcore_axis_name="s")


@functools.partial(
    pl.kernel,
    out_type=jax.ShapeDtypeStruct((BATCH,), jnp.float32),
    mesh=_mesh,
    compiler_params=pltpu.CompilerParams(
        needs_layout_passes=False, use_tc_tiling_on_sc=True
    ),
    scratch_types=[
        pltpu.VMEM((BPW + 16,), jnp.int32),
        pltpu.VMEM((BPW + 16,), jnp.int32),
        pltpu.VMEM((KBLK, N_FACTORS, 128), jnp.float32),
        pltpu.VMEM((KBLK, N_FACTORS, 128), jnp.float32),
        pltpu.VMEM((BPW,), jnp.float32),
        pltpu.SemaphoreType.DMA,
        pltpu.SemaphoreType.DMA,
    ],
)
def _sc_dot_kernel(users_hbm, items_hbm, ut_hbm, vt_hbm, out_hbm,
                   idx_u, idx_i, ublk, vblk, outv, sem_u, sem_v):
    wid = lax.axis_index("s") * NC + lax.axis_index("c")
    base = wid * BPW
    pltpu.sync_copy(users_hbm.at[pl.ds(base, BPW)],
                    idx_u.at[pl.ds(0, BPW)])
    pltpu.sync_copy(items_hbm.at[pl.ds(base, BPW)],
                    idx_i.at[pl.ds(0, BPW)])
    lane = lax.iota(jnp.int32, 16)
    last_lane = lane == 15
    d_lo = lane
    d_hi = lane + 16

    def blk_body(blk, _):
        uvec = idx_u[pl.ds(blk * KBLK, 16)]
        ivec = idx_i[pl.ds(blk * KBLK, 16)]
        for b in range(KBLK):
            iu = uvec[b]
            ii = ivec[b]
            off_u = pl.multiple_of((iu >> 7) * 128, 128)
            off_i = pl.multiple_of((ii >> 7) * 128, 128)
            pltpu.async_copy(ut_hbm.at[:, pl.ds(off_u, 128)],
                             ublk.at[b], sem_u)
            pltpu.async_copy(vt_hbm.at[:, pl.ds(off_i, 128)],
                             vblk.at[b], sem_v)
        for b in range(KBLK):
            pltpu.make_async_copy(ut_hbm.at[:, pl.ds(0, 128)],
                                  ublk.at[0], sem_u).wait()
            pltpu.make_async_copy(vt_hbm.at[:, pl.ds(0, 128)],
                                  vblk.at[0], sem_v).wait()
        for b in range(KBLK):
            lu = jnp.full((16,), uvec[b] & 127, jnp.int32)
            li = jnp.full((16,), ivec[b] & 127, jnp.int32)
            bb = jnp.full((16,), b, jnp.int32)
            u0 = plsc.load_gather(ublk, [bb, d_lo, lu])
            u1 = plsc.load_gather(ublk, [bb, d_hi, lu])
            v0 = plsc.load_gather(vblk, [bb, d_lo, li])
            v1 = plsc.load_gather(vblk, [bb, d_hi, li])
            s = plsc.cumsum(u0 * v0 + u1 * v1)
            pos = jnp.full((16,), blk * KBLK + b, jnp.int32)
            plsc.store_scatter(outv, [pos], s, mask=last_lane)
        return 0

    lax.fori_loop(0, NBLK, blk_body, 0)
    pltpu.sync_copy(outv, out_hbm.at[pl.ds(base, BPW)])


def kernel(data, user_factors, item_factors):
    users = data[:, 0].astype(jnp.int32)
    items = data[:, 1].astype(jnp.int32)
    return _sc_dot_kernel(users, items, user_factors.T, item_factors.T)


# speedup vs baseline: 3.8824x; 1.1422x over previous
import functools

import jax
import jax.numpy as jnp
from jax import lax
from jax.experimental import pallas as pl
from jax.experimental.pallas import tpu as pltpu
from jax.experimental.pallas import tpu_sc as plsc

N_FACTORS = 32
BATCH = 16384
NC = 2
NS = 16
NW = NC * NS
BPW = BATCH // NW
HALF = 4
STEP = 2 * HALF
NSTEP = BPW // STEP

_mesh = plsc.VectorSubcoreMesh(core_axis_name="c", subcore_axis_name="s")


@functools.partial(
    pl.kernel,
    out_type=jax.ShapeDtypeStruct((BATCH,), jnp.float32),
    mesh=_mesh,
    compiler_params=pltpu.CompilerParams(
        needs_layout_passes=False, use_tc_tiling_on_sc=True
    ),
    scratch_types=[
        pltpu.VMEM((BPW + 16,), jnp.int32),
        pltpu.VMEM((BPW + 16,), jnp.int32),
        pltpu.VMEM((N_FACTORS, STEP * 128), jnp.float32),
        pltpu.VMEM((N_FACTORS, STEP * 128), jnp.float32),
        pltpu.VMEM((BPW,), jnp.float32),
        pltpu.SemaphoreType.DMA,
        pltpu.SemaphoreType.DMA,
        pltpu.SemaphoreType.DMA,
        pltpu.SemaphoreType.DMA,
    ],
)
def _sc_dot_kernel(users_hbm, items_hbm, ut_hbm, vt_hbm, out_hbm,
                   idx_u, idx_i, ublk, vblk, outv,
                   sem_ua, sem_ub, sem_va, sem_vb):
    wid = lax.axis_index("s") * NC + lax.axis_index("c")
    base = wid * BPW
    pltpu.sync_copy(users_hbm.at[pl.ds(base, BPW)],
                    idx_u.at[pl.ds(0, BPW)])
    pltpu.sync_copy(items_hbm.at[pl.ds(base, BPW)],
                    idx_i.at[pl.ds(0, BPW)])
    lane = lax.iota(jnp.int32, 16)
    last_lane = lane == 15
    d_lo = lane
    d_hi = lane + 16

    def issue_half(uvec, ivec, j0, slot0, sem_u, sem_v):
        for j in range(HALF):
            iu = uvec[j0 + j]
            ii = ivec[j0 + j]
            off_u = pl.multiple_of((iu >> 7) * 128, 128)
            off_i = pl.multiple_of((ii >> 7) * 128, 128)
            col = (slot0 + j) * 128
            pltpu.async_copy(ut_hbm.at[:, pl.ds(off_u, 128)],
                             ublk.at[:, pl.ds(col, 128)], sem_u)
            pltpu.async_copy(vt_hbm.at[:, pl.ds(off_i, 128)],
                             vblk.at[:, pl.ds(col, 128)], sem_v)

    def drain_half(slot0, sem_u, sem_v):
        col = slot0 * 128
        pltpu.make_async_copy(
            ut_hbm.at[:, pl.ds(0, HALF * 128)],
            ublk.at[:, pl.ds(col, HALF * 128)], sem_u).wait()
        pltpu.make_async_copy(
            vt_hbm.at[:, pl.ds(0, HALF * 128)],
            vblk.at[:, pl.ds(col, HALF * 128)], sem_v).wait()

    def compute_half(uvec, ivec, j0, slot0, e0):
        for j in range(HALF):
            cbase = (slot0 + j) * 128
            lu = cbase + jnp.full((16,), uvec[j0 + j] & 127, jnp.int32)
            li = cbase + jnp.full((16,), ivec[j0 + j] & 127, jnp.int32)
            u0 = plsc.load_gather(ublk, [d_lo, lu])
            u1 = plsc.load_gather(ublk, [d_hi, lu])
            v0 = plsc.load_gather(vblk, [d_lo, li])
            v1 = plsc.load_gather(vblk, [d_hi, li])
            s = plsc.cumsum(u0 * v0 + u1 * v1)
            pos = jnp.full((16,), e0 + j, jnp.int32)
            plsc.store_scatter(outv, [pos], s, mask=last_lane)

    uvec0 = idx_u[pl.ds(0, 16)]
    ivec0 = idx_i[pl.ds(0, 16)]
    issue_half(uvec0, ivec0, 0, 0, sem_ua, sem_va)

    def step(q, _):
        e0 = q * STEP
        uvec = idx_u[pl.ds(e0, 16)]
        ivec = idx_i[pl.ds(e0, 16)]
        issue_half(uvec, ivec, HALF, HALF, sem_ub, sem_vb)
        drain_half(0, sem_ua, sem_va)
        compute_half(uvec, ivec, 0, 0, e0)

        @pl.when(q < NSTEP - 1)
        def _():
            uvec_n = idx_u[pl.ds(e0 + STEP, 16)]
            ivec_n = idx_i[pl.ds(e0 + STEP, 16)]
            issue_half(uvec_n, ivec_n, 0, 0, sem_ua, sem_va)

        drain_half(HALF, sem_ub, sem_vb)
        compute_half(uvec, ivec, HALF, HALF, e0 + HALF)
        return 0

    lax.fori_loop(0, NSTEP, step, 0)
    pltpu.sync_copy(outv, out_hbm.at[pl.ds(base, BPW)])


def kernel(data, user_factors, item_factors):
    users = data[:, 0].astype(jnp.int32)
    items = data[:, 1].astype(jnp.int32)
    return _sc_dot_kernel(users, items, user_factors.T, item_factors.T)
